# Initial kernel scaffold; baseline (speedup 1.0000x reference)
import jax
import jax.numpy as jnp
from jax.experimental import pallas as pl


def kernel(q, k, v):
    raise NotImplementedError("write your pallas kernel here")


# speedup vs baseline: 30.0846x; 30.0846x over previous
import math

import jax
import jax.numpy as jnp
from jax.experimental import pallas as pl
from jax.experimental.pallas import tpu as pltpu

N_HEADS = 16
D_MODEL = 1024
D_HEAD = D_MODEL // N_HEADS
TOP_K = 64
CONTEXT_LEN = 2048
NEG_INF = -1e30


def _rope_tables_full(T, d_head):
    position = jnp.arange(T, dtype=jnp.float32)[:, None]
    div_term = 10000.0 ** (jnp.arange(0, d_head, 2, dtype=jnp.float32) / d_head)
    div_term = jnp.repeat(div_term, 2)
    cos = jnp.cos(position / div_term)
    sin = jnp.sin(position / div_term)
    return cos, sin


def _pair_swap_matrix(d_head):
    import numpy as np

    P = np.zeros((d_head, d_head), dtype=np.float32)
    for i in range(d_head // 2):
        P[2 * i + 1, 2 * i] = -1.0
        P[2 * i, 2 * i + 1] = 1.0
    return jnp.asarray(P)


def _attn_kernel(cos_ref, sin_ref, perm_ref, q_ref, k_ref, v_ref, o_ref,
                 kr_ref, *, bq, T, top_k):
    qi = pl.program_id(1)
    scale = 1.0 / math.sqrt(D_HEAD)

    P = perm_ref[...]
    cos = cos_ref[...]
    sin = sin_ref[...]

    @pl.when(qi == 0)
    def _():
        kh = k_ref[0]
        kr_ref[...] = kh * cos + jax.lax.dot(
            kh, P, preferred_element_type=jnp.float32,
            precision=jax.lax.Precision.HIGHEST) * sin

    qh = q_ref[0]
    qpos = qi * bq
    cq = cos_ref[pl.ds(qpos, bq), :]
    sq = sin_ref[pl.ds(qpos, bq), :]
    qr = qh * cq + jax.lax.dot(
        qh, P, preferred_element_type=jnp.float32,
        precision=jax.lax.Precision.HIGHEST) * sq

    kr = kr_ref[...]
    logits = jax.lax.dot_general(
        qr, kr, (((1,), (1,)), ((), ())),
        preferred_element_type=jnp.float32) * scale

    row = qpos + jax.lax.broadcasted_iota(jnp.int32, (bq, T), 0)
    col = jax.lax.broadcasted_iota(jnp.int32, (bq, T), 1)
    logits = jnp.where(col <= row, logits, NEG_INF)

    m = jnp.max(logits, axis=1, keepdims=True)

    b = jax.lax.bitcast_convert_type(logits, jnp.uint32)
    sign = jnp.uint32(0x80000000)
    u = jnp.where(b >= sign, ~b, b | sign)

    t = jnp.zeros((bq, 1), jnp.uint32)
    for i in range(31, -1, -1):
        cand = t | jnp.uint32(1 << i)
        cnt = jnp.sum((u >= cand).astype(jnp.float32), axis=1, keepdims=True)
        t = jnp.where(cnt >= float(top_k), cand, t)

    e = jnp.exp(logits - m)
    w = jnp.where(u >= t, e, 0.0)
    z_all = jnp.sum(e, axis=1, keepdims=True)
    denom = jnp.sum(w, axis=1, keepdims=True) + 1e-9 * z_all

    out = jax.lax.dot(w, v_ref[0], preferred_element_type=jnp.float32,
                      precision=jax.lax.Precision.HIGHEST) / denom
    o_ref[0] = out


def kernel(q, k, v):
    b, T, d_model = q.shape
    H, d_head = N_HEADS, D_HEAD
    assert b == 1 and d_model == D_MODEL

    qh = q.reshape(T, H, d_head).transpose(1, 0, 2)
    kh = k.reshape(T, H, d_head).transpose(1, 0, 2)
    vh = v.reshape(T, H, d_head).transpose(1, 0, 2)

    cos, sin = _rope_tables_full(CONTEXT_LEN, d_head)
    cos = cos[:T]
    sin = sin[:T]
    P = _pair_swap_matrix(d_head)

    bq = min(256, T)
    grid = (H, T // bq)

    from functools import partial

    out = pl.pallas_call(
        partial(_attn_kernel, bq=bq, T=T, top_k=TOP_K),
        grid=grid,
        in_specs=[
            pl.BlockSpec((T, d_head), lambda h, i: (0, 0)),
            pl.BlockSpec((T, d_head), lambda h, i: (0, 0)),
            pl.BlockSpec((d_head, d_head), lambda h, i: (0, 0)),
            pl.BlockSpec((1, bq, d_head), lambda h, i: (h, i, 0)),
            pl.BlockSpec((1, T, d_head), lambda h, i: (h, 0, 0)),
            pl.BlockSpec((1, T, d_head), lambda h, i: (h, 0, 0)),
        ],
        out_specs=pl.BlockSpec((1, bq, d_head), lambda h, i: (h, i, 0)),
        out_shape=jax.ShapeDtypeStruct((H, T, d_head), jnp.float32),
        scratch_shapes=[pltpu.VMEM((T, d_head), jnp.float32)],
        compiler_params=pltpu.CompilerParams(
            dimension_semantics=("arbitrary", "arbitrary")),
    )(cos, sin, P, qh, kh, vh)

    return out.transpose(1, 0, 2).reshape(1, T, d_model)
